# baseline (speedup 1.0000x reference)
import functools
import jax
import jax.numpy as jnp
from jax import lax
from jax.experimental import pallas as pl
from jax.experimental.pallas import tpu as pltpu
from jax.experimental.pallas import tpu_sc as plsc

B = 1024
L = 200
D = 64
V = 100000
HOPS = 3
NT = 4

NC = 2
NS = 16
NW = NC * NS
RPW = B // NW
LP = 208
NCH = LP // 16
IC = 2
ICL = LP // IC
NEG = -1e30


def _splat(x):
    return jnp.full((16,), x, jnp.int32)


def _bcast_lane(vec, lane):
    dn = lax.GatherDimensionNumbers(
        offset_dims=(), collapsed_slice_dims=(0,), start_index_map=(0,))
    idx = jnp.full((16, 1), lane, jnp.int32)
    return lax.gather(vec, idx, dn, slice_sizes=(1,),
                      mode=lax.GatherScatterMode.PROMISE_IN_BOUNDS)


def _sc_attn_body(ids_hbm, hid_hbm, t0, t1, t2, t3, u_hbm,
                  gbuf, idxb, ubuf, lbuf, ebuf, sem0, sem1):
    tables = [t0, t1, t2, t3]
    sems = [sem0, sem1]
    iota16 = lax.iota(jnp.int32, 16)
    lane0 = iota16 == 0
    wid = lax.axis_index("s") * NC + lax.axis_index("c")
    row0 = wid * RPW

    def fire(row, slot):
        pltpu.sync_copy(ids_hbm.at[row], idxb.at[slot])
        for t in range(NT):
            for c in range(IC):
                pltpu.async_copy(
                    tables[t].at[idxb.at[slot, c]],
                    gbuf.at[slot, t, pl.ds(c * ICL, ICL)],
                    sems[slot])

    def drain(row, slot):
        for t in range(NT):
            for c in range(IC):
                pltpu.make_async_copy(
                    tables[t].at[idxb.at[slot, c]],
                    gbuf.at[slot, t, pl.ds(c * ICL, ICL)],
                    sems[slot]).wait()

    def compute(row, slot):
        pltpu.sync_copy(hid_hbm.at[row], ubuf)

        def hop_body(h, _):
            up = [ubuf[pl.ds(16 * k, 16)] for k in range(4)]
            th = _splat(h)
            slotv = _splat(slot)

            def logit_c(c, _c):
                lvec = c * 16 + iota16
                acc = jnp.zeros((16,), jnp.float32)
                for d in range(D):
                    ub = _bcast_lane(up[d // 16], d % 16)
                    g = plsc.load_gather(gbuf, [slotv, th, lvec, _splat(d)])
                    acc = acc + ub * g
                acc = jnp.where(lvec < L, acc, NEG)
                lbuf[pl.ds(c * 16, 16)] = acc
                return 0

            lax.fori_loop(0, NCH, logit_c, 0)

            mv = lbuf[pl.ds(0, 16)]
            for c in range(1, NCH):
                mv = jnp.maximum(mv, lbuf[pl.ds(c * 16, 16)])
            m = jnp.max(mv)
            sacc = jnp.zeros((16,), jnp.float32)
            es = []
            for c in range(NCH):
                e = jnp.exp(lbuf[pl.ds(c * 16, 16)] - m)
                ebuf[pl.ds(c * 16, 16)] = e
                es.append(e)
                sacc = sacc + e
            sv = jnp.zeros((16,), jnp.float32) + jnp.sum(sacc)
            inv = jnp.ones((16,), jnp.float32) / sv
            lvecs = [c * 16 + iota16 for c in range(NCH)]
            th1 = _splat(h + 1)

            def o_d(d, _d):
                dv = _splat(d)
                acc = jnp.zeros((16,), jnp.float32)
                for c in range(NCH):
                    g = plsc.load_gather(gbuf, [slotv, th1, lvecs[c], dv])
                    acc = acc + es[c] * g
                od = jnp.sum(acc)
                ubd = plsc.load_gather(ubuf, [dv])
                plsc.store_scatter(ubuf, [dv], ubd + inv * od, mask=lane0)
                return 0

            lax.fori_loop(0, D, o_d, 0)
            return 0

        lax.fori_loop(0, HOPS, hop_body, 0)
        pltpu.sync_copy(ubuf, u_hbm.at[row])

    fire(row0, 0)

    def pair_body(i, _):
        r = row0 + 2 * i
        for s in (0, 1):
            row = r + s
            nxt = row + 1

            @pl.when(nxt < row0 + RPW)
            def _():
                fire(nxt, 1 - s)

            drain(row, s)
            compute(row, s)
        return 0

    lax.fori_loop(0, RPW // 2, pair_body, 0)


@jax.jit
def _sc_attention(ids3, hidden, t0, t1, t2, t3):
    mesh = plsc.VectorSubcoreMesh(core_axis_name="c", subcore_axis_name="s")
    return pl.kernel(
        _sc_attn_body,
        out_type=jax.ShapeDtypeStruct((B, D), jnp.float32),
        mesh=mesh,
        scratch_types=[
            pltpu.VMEM((2, NT, LP, D), jnp.float32),
            pltpu.VMEM((2, IC, ICL), jnp.int32),
            pltpu.VMEM((D,), jnp.float32),
            pltpu.VMEM((LP,), jnp.float32),
            pltpu.VMEM((LP,), jnp.float32),
            pltpu.SemaphoreType.DMA,
            pltpu.SemaphoreType.DMA,
        ],
        compiler_params=pltpu.CompilerParams(
            use_tc_tiling_on_sc=False, needs_layout_passes=False),
    )(ids3, hidden, t0, t1, t2, t3)


VB = 2048


def _classifier_body(u_ref, w_ref, b_ref, o_ref):
    acc = lax.dot_general(
        u_ref[...], w_ref[...],
        dimension_numbers=(((1,), (1,)), ((), ())),
        preferred_element_type=jnp.float32,
    )
    o_ref[...] = jax.nn.sigmoid(acc + b_ref[...])


@jax.jit
def _classifier(u, W, b2):
    nvb = pl.cdiv(V, VB)
    return pl.pallas_call(
        _classifier_body,
        grid=(nvb,),
        in_specs=[
            pl.BlockSpec((B, D), lambda j: (0, 0)),
            pl.BlockSpec((VB, D), lambda j: (j, 0)),
            pl.BlockSpec((1, VB), lambda j: (0, j)),
        ],
        out_specs=pl.BlockSpec((B, VB), lambda j: (0, j)),
        out_shape=jax.ShapeDtypeStruct((B, V), jnp.float32),
    )(u, W, b2)


def kernel(input_ids, hidden_states, C0, C1, C2, C3, W, b):
    ids = input_ids.astype(jnp.int32)
    ids_pad = jnp.pad(ids, ((0, 0), (0, LP - L))).reshape(B, IC, ICL)
    u = _sc_attention(ids_pad, hidden_states, C0, C1, C2, C3)
    return _classifier(u, W, b.reshape(1, V))


# speedup vs baseline: 1.5027x; 1.5027x over previous
import functools
import jax
import jax.numpy as jnp
from jax import lax
from jax.experimental import pallas as pl
from jax.experimental.pallas import tpu as pltpu
from jax.experimental.pallas import tpu_sc as plsc

B = 1024
L = 200
D = 64
V = 100000
HOPS = 3
NT = 4

NC = 2
NS = 16
NW = NC * NS
RPW = B // NW
LP = 208
NCH = LP // 16
IC = 2
ICL = LP // IC
NEG = -1e30


def _splat(x):
    return jnp.full((16,), x, jnp.int32)


def _bcast_lane(vec, lane):
    dn = lax.GatherDimensionNumbers(
        offset_dims=(), collapsed_slice_dims=(0,), start_index_map=(0,))
    idx = jnp.full((16, 1), lane, jnp.int32)
    return lax.gather(vec, idx, dn, slice_sizes=(1,),
                      mode=lax.GatherScatterMode.PROMISE_IN_BOUNDS)


def _sc_attn_body(ids_hbm, hid_hbm, t0, t1, t2, t3, u_hbm,
                  gbuf, idxb, ubuf, lbuf, ebuf, sem0, sem1):
    tables = [t0, t1, t2, t3]
    sems = [sem0, sem1]
    iota16 = lax.iota(jnp.int32, 16)
    lane0 = iota16 == 0
    wid = lax.axis_index("s") * NC + lax.axis_index("c")
    row0 = wid * RPW

    def fire(row, slot):
        pltpu.sync_copy(ids_hbm.at[row], idxb.at[slot])
        for t in range(NT):
            for c in range(IC):
                pltpu.async_copy(
                    tables[t].at[idxb.at[slot, c]],
                    gbuf.at[slot, t, pl.ds(c * ICL, ICL)],
                    sems[slot])

    def drain(row, slot):
        for t in range(NT):
            for c in range(IC):
                pltpu.make_async_copy(
                    tables[t].at[idxb.at[slot, c]],
                    gbuf.at[slot, t, pl.ds(c * ICL, ICL)],
                    sems[slot]).wait()

    def compute(row, slot):
        pltpu.sync_copy(hid_hbm.at[row], ubuf)

        def hop_body(h, _):
            up = [ubuf[pl.ds(16 * k, 16)] for k in range(4)]
            th = _splat(h)
            slotv = _splat(slot)
            zero = jnp.zeros((16,), jnp.float32)

            def logit_c(c, _c):
                lvec = c * 16 + iota16
                accs = [zero, zero, zero, zero]
                for d in range(D):
                    ub = _bcast_lane(up[d // 16], d % 16)
                    g = plsc.load_gather(gbuf, [slotv, th, lvec, _splat(d)])
                    accs[d % 4] = accs[d % 4] + ub * g
                acc = (accs[0] + accs[1]) + (accs[2] + accs[3])
                acc = jnp.where(lvec < L, acc, NEG)
                lbuf[pl.ds(c * 16, 16)] = acc
                return 0

            lax.fori_loop(0, NCH, logit_c, 0)

            mv = lbuf[pl.ds(0, 16)]
            for c in range(1, NCH):
                mv = jnp.maximum(mv, lbuf[pl.ds(c * 16, 16)])
            m = jnp.max(mv)
            sacc = jnp.zeros((16,), jnp.float32)
            for c in range(NCH):
                e = jnp.exp(lbuf[pl.ds(c * 16, 16)] - m)
                ebuf[pl.ds(c * 16, 16)] = e
                sacc = sacc + e
            sv = jnp.zeros((16,), jnp.float32) + jnp.sum(sacc)
            inv = jnp.ones((16,), jnp.float32) / sv

            def o_c(c, accs):
                e_c = ebuf[pl.ds(c * 16, 16)]
                new = list(accs)
                for j in range(16):
                    eb = _bcast_lane(e_c, j)
                    lrow = c * 16 + j
                    for k in range(4):
                        g = gbuf[slot, h + 1, lrow, pl.ds(16 * k, 16)]
                        a = k * 2 + (j % 2)
                        new[a] = new[a] + eb * g
                return tuple(new)

            accs = lax.fori_loop(0, NCH, o_c, (zero,) * 8)
            for k in range(4):
                ok = accs[k * 2] + accs[k * 2 + 1]
                ubuf[pl.ds(16 * k, 16)] = up[k] + inv * ok
            return 0

        lax.fori_loop(0, HOPS, hop_body, 0)
        pltpu.sync_copy(ubuf, u_hbm.at[row])

    fire(row0, 0)

    def pair_body(i, _):
        r = row0 + 2 * i
        for s in (0, 1):
            row = r + s
            nxt = row + 1

            @pl.when(nxt < row0 + RPW)
            def _():
                fire(nxt, 1 - s)

            drain(row, s)
            compute(row, s)
        return 0

    lax.fori_loop(0, RPW // 2, pair_body, 0)


@jax.jit
def _sc_attention(ids3, hidden, t0, t1, t2, t3):
    mesh = plsc.VectorSubcoreMesh(core_axis_name="c", subcore_axis_name="s")
    return pl.kernel(
        _sc_attn_body,
        out_type=jax.ShapeDtypeStruct((B, D), jnp.float32),
        mesh=mesh,
        scratch_types=[
            pltpu.VMEM((2, NT, LP, D), jnp.float32),
            pltpu.VMEM((2, IC, ICL), jnp.int32),
            pltpu.VMEM((D,), jnp.float32),
            pltpu.VMEM((LP,), jnp.float32),
            pltpu.VMEM((LP,), jnp.float32),
            pltpu.SemaphoreType.DMA,
            pltpu.SemaphoreType.DMA,
        ],
        compiler_params=pltpu.CompilerParams(
            use_tc_tiling_on_sc=False, needs_layout_passes=False),
    )(ids3, hidden, t0, t1, t2, t3)


VB = 2048


def _classifier_body(u_ref, w_ref, b_ref, o_ref):
    acc = lax.dot_general(
        u_ref[...], w_ref[...],
        dimension_numbers=(((1,), (1,)), ((), ())),
        preferred_element_type=jnp.float32,
    )
    o_ref[...] = jax.nn.sigmoid(acc + b_ref[...])


@jax.jit
def _classifier(u, W, b2):
    nvb = pl.cdiv(V, VB)
    return pl.pallas_call(
        _classifier_body,
        grid=(nvb,),
        in_specs=[
            pl.BlockSpec((B, D), lambda j: (0, 0)),
            pl.BlockSpec((VB, D), lambda j: (j, 0)),
            pl.BlockSpec((1, VB), lambda j: (0, j)),
        ],
        out_specs=pl.BlockSpec((B, VB), lambda j: (0, j)),
        out_shape=jax.ShapeDtypeStruct((B, V), jnp.float32),
    )(u, W, b2)


def kernel(input_ids, hidden_states, C0, C1, C2, C3, W, b):
    ids = input_ids.astype(jnp.int32)
    ids_pad = jnp.pad(ids, ((0, 0), (0, LP - L))).reshape(B, IC, ICL)
    u = _sc_attention(ids_pad, hidden_states, C0, C1, C2, C3)
    return _classifier(u, W, b.reshape(1, V))


# speedup vs baseline: 2.1459x; 1.4280x over previous
import functools
import jax
import jax.numpy as jnp
from jax import lax
from jax.experimental import pallas as pl
from jax.experimental.pallas import tpu as pltpu
from jax.experimental.pallas import tpu_sc as plsc

B = 1024
L = 200
D = 64
V = 100000
HOPS = 3
NT = 4

NC = 2
NS = 16
NW = NC * NS
RPW = B // NW
LP = 208
NCH = LP // 16
IC = 2
ICL = LP // IC
NEG = -1e30


def _splat(x):
    return jnp.full((16,), x, jnp.int32)


def _bcast_lane(vec, lane):
    dn = lax.GatherDimensionNumbers(
        offset_dims=(), collapsed_slice_dims=(0,), start_index_map=(0,))
    idx = jnp.full((16, 1), lane, jnp.int32)
    return lax.gather(vec, idx, dn, slice_sizes=(1,),
                      mode=lax.GatherScatterMode.PROMISE_IN_BOUNDS)


def _sc_attn_body(ids_hbm, hid_hbm, t0, t1, t2, t3, u_hbm,
                  gbuf, idxb, ubuf, lbuf, ebuf, sem0, sem1):
    tables = [t0, t1, t2, t3]
    sems = [sem0, sem1]
    iota16 = lax.iota(jnp.int32, 16)
    lane0 = iota16 == 0
    wid = lax.axis_index("s") * NC + lax.axis_index("c")
    row0 = wid * RPW

    def fire(row, slot):
        pltpu.sync_copy(ids_hbm.at[row], idxb.at[slot])
        for t in range(NT):
            for c in range(IC):
                pltpu.async_copy(
                    tables[t].at[idxb.at[slot, c]],
                    gbuf.at[slot, t, pl.ds(c * ICL, ICL)],
                    sems[slot])

    def drain(row, slot):
        for t in range(NT):
            for c in range(IC):
                pltpu.make_async_copy(
                    tables[t].at[idxb.at[slot, c]],
                    gbuf.at[slot, t, pl.ds(c * ICL, ICL)],
                    sems[slot]).wait()

    def compute(row, slot):
        pltpu.sync_copy(hid_hbm.at[row], ubuf)

        def hop_body(h, _):
            up = [ubuf[pl.ds(16 * k, 16)] for k in range(4)]
            zero = jnp.zeros((16,), jnp.float32)

            def logit_c(c, _c):
                lvec = c * 16 + iota16
                lacc = zero
                for j in range(16):
                    lrow = c * 16 + j
                    p01 = (up[0] * gbuf[slot, h, lrow, pl.ds(0, 16)]
                           + up[1] * gbuf[slot, h, lrow, pl.ds(16, 16)])
                    p23 = (up[2] * gbuf[slot, h, lrow, pl.ds(32, 16)]
                           + up[3] * gbuf[slot, h, lrow, pl.ds(48, 16)])
                    cs = plsc.cumsum(p01 + p23)
                    sv = _bcast_lane(cs, 15)
                    lacc = jnp.where(iota16 == j, sv, lacc)
                lacc = jnp.where(lvec < L, lacc, NEG)
                lbuf[pl.ds(c * 16, 16)] = lacc
                return 0

            lax.fori_loop(0, NCH, logit_c, 0)

            mv = lbuf[pl.ds(0, 16)]
            for c in range(1, NCH):
                mv = jnp.maximum(mv, lbuf[pl.ds(c * 16, 16)])
            m = jnp.max(mv)
            sacc = jnp.zeros((16,), jnp.float32)
            for c in range(NCH):
                e = jnp.exp(lbuf[pl.ds(c * 16, 16)] - m)
                ebuf[pl.ds(c * 16, 16)] = e
                sacc = sacc + e
            sv = jnp.zeros((16,), jnp.float32) + jnp.sum(sacc)
            inv = jnp.ones((16,), jnp.float32) / sv

            def o_c(c, accs):
                e_c = ebuf[pl.ds(c * 16, 16)]
                new = list(accs)
                for j in range(16):
                    eb = _bcast_lane(e_c, j)
                    lrow = c * 16 + j
                    for k in range(4):
                        g = gbuf[slot, h + 1, lrow, pl.ds(16 * k, 16)]
                        a = k * 2 + (j % 2)
                        new[a] = new[a] + eb * g
                return tuple(new)

            accs = lax.fori_loop(0, NCH, o_c, (zero,) * 8)
            for k in range(4):
                ok = accs[k * 2] + accs[k * 2 + 1]
                ubuf[pl.ds(16 * k, 16)] = up[k] + inv * ok
            return 0

        lax.fori_loop(0, HOPS, hop_body, 0)
        pltpu.sync_copy(ubuf, u_hbm.at[row])

    fire(row0, 0)

    def pair_body(i, _):
        r = row0 + 2 * i
        for s in (0, 1):
            row = r + s
            nxt = row + 1

            @pl.when(nxt < row0 + RPW)
            def _():
                fire(nxt, 1 - s)

            drain(row, s)
            compute(row, s)
        return 0

    lax.fori_loop(0, RPW // 2, pair_body, 0)


@jax.jit
def _sc_attention(ids3, hidden, t0, t1, t2, t3):
    mesh = plsc.VectorSubcoreMesh(core_axis_name="c", subcore_axis_name="s")
    return pl.kernel(
        _sc_attn_body,
        out_type=jax.ShapeDtypeStruct((B, D), jnp.float32),
        mesh=mesh,
        scratch_types=[
            pltpu.VMEM((2, NT, LP, D), jnp.float32),
            pltpu.VMEM((2, IC, ICL), jnp.int32),
            pltpu.VMEM((D,), jnp.float32),
            pltpu.VMEM((LP,), jnp.float32),
            pltpu.VMEM((LP,), jnp.float32),
            pltpu.SemaphoreType.DMA,
            pltpu.SemaphoreType.DMA,
        ],
        compiler_params=pltpu.CompilerParams(
            use_tc_tiling_on_sc=False, needs_layout_passes=False),
    )(ids3, hidden, t0, t1, t2, t3)


VB = 2048


def _classifier_body(u_ref, w_ref, b_ref, o_ref):
    acc = lax.dot_general(
        u_ref[...], w_ref[...],
        dimension_numbers=(((1,), (1,)), ((), ())),
        preferred_element_type=jnp.float32,
    )
    o_ref[...] = jax.nn.sigmoid(acc + b_ref[...])


@jax.jit
def _classifier(u, W, b2):
    nvb = pl.cdiv(V, VB)
    return pl.pallas_call(
        _classifier_body,
        grid=(nvb,),
        in_specs=[
            pl.BlockSpec((B, D), lambda j: (0, 0)),
            pl.BlockSpec((VB, D), lambda j: (j, 0)),
            pl.BlockSpec((1, VB), lambda j: (0, j)),
        ],
        out_specs=pl.BlockSpec((B, VB), lambda j: (0, j)),
        out_shape=jax.ShapeDtypeStruct((B, V), jnp.float32),
    )(u, W, b2)


def kernel(input_ids, hidden_states, C0, C1, C2, C3, W, b):
    ids = input_ids.astype(jnp.int32)
    ids_pad = jnp.pad(ids, ((0, 0), (0, LP - L))).reshape(B, IC, ICL)
    u = _sc_attention(ids_pad, hidden_states, C0, C1, C2, C3)
    return _classifier(u, W, b.reshape(1, V))
